# Initial kernel scaffold; baseline (speedup 1.0000x reference)
import jax
import jax.numpy as jnp
from jax.experimental import pallas as pl


def kernel(data):
    raise NotImplementedError("write your pallas kernel here")


# speedup vs baseline: 2.6375x; 2.6375x over previous
import jax
import jax.numpy as jnp
from jax.experimental import pallas as pl

_T = 32
_LANES = 128


def _rank_kernel(x_ref, out_ref):
    a = jnp.abs(x_ref[0])
    r, c = a.shape
    lin = (jax.lax.broadcasted_iota(jnp.int32, (r, c), 0) * c
           + jax.lax.broadcasted_iota(jnp.int32, (r, c), 1))

    def body(t, a):
        m = jnp.max(a)
        idx = jnp.min(jnp.where(a == m, lin, jnp.int32(r * c)))
        onehot = lin == idx
        out_ref[0, t] = onehot.astype(jnp.float32)
        return jnp.where(onehot, jnp.float32(-1.0), a)

    a = jax.lax.fori_loop(0, _T - 1, body, a)
    out_ref[0, _T - 1] = (a >= 0).astype(jnp.float32)


def kernel(data):
    b, n = data.shape
    r = n // _LANES
    x = data.reshape(b, r, _LANES)
    out = pl.pallas_call(
        _rank_kernel,
        grid=(b,),
        in_specs=[pl.BlockSpec((1, r, _LANES), lambda i: (i, 0, 0))],
        out_specs=pl.BlockSpec((1, _T, r, _LANES), lambda i: (i, 0, 0, 0)),
        out_shape=jax.ShapeDtypeStruct((b, _T, r, _LANES), jnp.float32),
    )(x)
    return out.reshape(b, _T, n)
